# Initial kernel scaffold; baseline (speedup 1.0000x reference)
import jax
import jax.numpy as jnp
from jax.experimental import pallas as pl


def kernel(x):
    raise NotImplementedError("write your pallas kernel here")


# speedup vs baseline: 31.9424x; 31.9424x over previous
import functools

import jax
import jax.numpy as jnp
from jax import lax
from jax.experimental import pallas as pl
from jax.experimental.pallas import tpu as pltpu

_H = 512
_W = 512
_N = _H * _W
_KLO = 26214
_KMAX = 26214
_KHI = _N - _KMAX + 1
_IGNORE = 255


def _select_body(x_ref, out_ref):
    x = x_ref[0, 0]
    s = lax.bitcast_convert_type(x, jnp.int32)
    key = jnp.where(s >= 0, s, s ^ jnp.int32(0x7FFFFFFF))
    key = jnp.where(s == jnp.int32(-2147483648), jnp.int32(0), key)

    def step(t, carry):
        qlo, qhi = carry
        bit = jnp.left_shift(jnp.int32(1), jnp.int32(31) - t)
        plo = qlo + bit
        phi = qhi + bit
        clo = jnp.sum((key < plo).astype(jnp.int32))
        chi = jnp.sum((key < phi).astype(jnp.int32))
        qlo = jnp.where(clo < _KLO, plo, qlo)
        qhi = jnp.where(chi < _KHI, phi, qhi)
        return qlo, qhi

    qinit = jnp.int32(-2147483648)
    qlo, qhi = lax.fori_loop(0, 32, step, (qinit, qinit))

    m_lo = jnp.sum((key < qlo).astype(jnp.int32))
    m_hi = jnp.sum((key < qhi).astype(jnp.int32))

    eq_lo = (key == qlo)
    eq_hi = (key == qhi)

    r_iota = lax.broadcasted_iota(jnp.int32, (_H, _W), 0)
    c_iota = lax.broadcasted_iota(jnp.int32, (_H, _W), 1)
    upper = (r_iota < c_iota).astype(jnp.bfloat16)
    lower = (c_iota < r_iota).astype(jnp.float32)

    dn = (((1,), (0,)), ((), ()))

    def prefix_of(eq):
        eq_b = eq.astype(jnp.bfloat16)
        within = lax.dot_general(eq_b, upper, dn,
                                 preferred_element_type=jnp.float32)
        rowsum = jnp.sum(eq.astype(jnp.float32), axis=1, keepdims=True)
        row_prefix = lax.dot_general(lower, rowsum, dn,
                                     preferred_element_type=jnp.float32)
        return within + row_prefix

    pref_lo = prefix_of(eq_lo)
    pref_hi = prefix_of(eq_hi)

    t_lo = (_KLO - m_lo).astype(jnp.float32)
    t_hi = (jnp.int32(_N) - m_hi - jnp.int32(_KMAX)).astype(jnp.float32)
    bg = (key < qlo) | (eq_lo & (pref_lo < t_lo))
    fg = (key > qhi) | (eq_hi & (pref_hi >= t_hi))

    out = jnp.where(fg, jnp.int32(1),
                    jnp.where(bg, jnp.int32(0), jnp.int32(_IGNORE)))
    out_ref[0] = out


@jax.jit
def kernel(x):
    b = x.shape[0]
    grid_spec = pl.GridSpec(
        grid=(b,),
        in_specs=[pl.BlockSpec((1, 1, _H, _W), lambda i: (i, 0, 0, 0))],
        out_specs=pl.BlockSpec((1, _H, _W), lambda i: (i, 0, 0)),
    )
    return pl.pallas_call(
        _select_body,
        grid_spec=grid_spec,
        out_shape=jax.ShapeDtypeStruct((b, _H, _W), jnp.int32),
    )(x)
